# baseline (speedup 1.0000x reference)
import functools
import math

import jax
import jax.numpy as jnp
import numpy as np
from jax import lax
from jax.experimental import pallas as pl
from jax.experimental.pallas import tpu as pltpu
from jax.experimental.pallas import tpu_sc as plsc

N_NODES = 10000
N_EDGES = 320000
HID = 128
AVG_LOG = float(np.log(33.0))
BLK = 512

NSUB = 32
NPS = 320
N_PAD = 10016
OFFS_PAD = 10264
C_EDGES = 128
SRC_PAD = N_EDGES + 256
MAXCH = (N_EDGES + C_EDGES + 8) // C_EDGES + 1
MAXCH_PAD = ((MAXCH + 23) // 8) * 8

STD_EPS_ROOT = float(math.sqrt(1e-5))


def _sc_agg_body(x_hbm, src_hbm, offs_hbm, chn_hbm, out_hbm,
                 offs_v, chn_v, idx_v0, idx_v1, rows_v0, rows_v1, stage,
                 sem0, sem1):
    w = lax.axis_index("s") * 2 + lax.axis_index("c")
    node_lo = w * NPS
    n_cnt = jnp.minimum(NPS, N_NODES - node_lo)
    pltpu.sync_copy(offs_hbm.at[pl.ds(node_lo, NPS + 24)], offs_v)
    pltpu.sync_copy(chn_hbm.at[w], chn_v)

    def offs_at(i):
        return offs_v[pl.ds(i, 16)]

    e_lo = offs_at(0)[0]
    e_hi = offs_at(n_cnt)[0]
    e8 = (e_lo // 8) * 8
    cnt = e_hi - e8
    nch = (cnt + C_EDGES - 1) // C_EDGES

    sems = (sem0, sem1)
    idxs = (idx_v0, idx_v1)
    rows = (rows_v0, rows_v1)

    def start(k, slot):
        pltpu.sync_copy(src_hbm.at[pl.ds(e8 + k * C_EDGES, C_EDGES)],
                        idxs[slot])
        pltpu.async_copy(x_hbm.at[idxs[slot]], rows[slot], sems[slot])

    def wait(slot):
        pltpu.make_async_copy(x_hbm.at[idxs[slot]], rows[slot],
                              sems[slot]).wait()

    def init_accs():
        z = jnp.zeros((16,), jnp.float32)
        ninf = jnp.full((16,), -jnp.inf, jnp.float32)
        pinf = jnp.full((16,), jnp.inf, jnp.float32)
        return (z,) * 8 + (z,) * 8 + (ninf,) * 8 + (pinf,) * 8

    def edge_loop(j0, j1, slot, accs):
        rv = rows[slot]

        def eb(j, a):
            out = []
            for c in range(8):
                r = rv[j, pl.ds(c * 16, 16)]
                out.append(a[c] + r)
            for c in range(8):
                r = rv[j, pl.ds(c * 16, 16)]
                out.append(a[8 + c] + r * r)
            for c in range(8):
                r = rv[j, pl.ds(c * 16, 16)]
                out.append(jnp.maximum(a[16 + c], r))
            for c in range(8):
                r = rv[j, pl.ds(c * 16, 16)]
                out.append(jnp.minimum(a[24 + c], r))
            return tuple(out)
        return lax.fori_loop(j0, jnp.maximum(j0, j1), eb, accs)

    def flush(np_, accs):
        row = np_ % 64
        for c in range(8):
            stage[row, pl.ds(c * 16, 16)] = accs[c]
            stage[row, pl.ds(128 + c * 16, 16)] = accs[24 + c]
            stage[row, pl.ds(256 + c * 16, 16)] = accs[16 + c]
            stage[row, pl.ds(384 + c * 16, 16)] = accs[8 + c]
        blk = np_ // 32

        @pl.when(np_ % 32 == 31)
        def _dma():
            pltpu.sync_copy(stage.at[pl.ds((blk % 2) * 32, 32)],
                            out_hbm.at[pl.ds(node_lo + blk * 32, 32)])

    def process(k, slot, carry):
        g0 = e8 + k * C_EDGES
        g1 = g0 + C_EDGES
        np0 = carry[0]
        np_end = chn_v[pl.ds(k, 16)][0]

        def nbody(np_, accs):
            v = offs_at(np_)
            a = jnp.maximum(v[0], g0)
            b = v[1]
            accs = edge_loop(a - g0, b - g0, slot, accs)
            flush(np_, accs)
            return init_accs()

        accs = lax.fori_loop(np0, jnp.maximum(np0, np_end), nbody, carry[1:])
        np_ = jnp.maximum(np0, np_end)
        v = offs_at(np_)
        a = jnp.maximum(v[0], g0)
        b = jnp.minimum(v[1], g1)
        b = jnp.where(np_ < n_cnt, b, a)
        accs = edge_loop(a - g0, b - g0, slot, accs)
        return (np_,) + accs

    @pl.when(nch > 0)
    def _prologue():
        start(0, 0)

    init = (jnp.int32(0),) + init_accs()

    def pair_body(p, carry):
        k0 = 2 * p

        @pl.when(k0 + 1 < nch)
        def _s1():
            start(k0 + 1, 1)

        wait(0)
        carry = process(k0, 0, carry)

        @pl.when(k0 + 2 < nch)
        def _s0():
            start(k0 + 2, 0)

        @pl.when(k0 + 1 < nch)
        def _w1():
            wait(1)

        carry = process(k0 + 1, 1, carry)
        return carry

    npairs = (nch + 1) // 2
    final = lax.fori_loop(0, npairs, pair_body, init)
    np_end = final[0]

    @pl.when(n_cnt % 32 != 0)
    def _tail_dma():
        blk = n_cnt // 32
        pltpu.sync_copy(stage.at[pl.ds((blk % 2) * 32, 32)],
                        out_hbm.at[pl.ds(node_lo + blk * 32, 32)])

    del np_end


@functools.cache
def _make_sc_agg(interpret=False):
    mesh = plsc.VectorSubcoreMesh(core_axis_name="c", subcore_axis_name="s",
                                  num_cores=2, num_subcores=16)
    return pl.kernel(
        _sc_agg_body,
        out_type=jax.ShapeDtypeStruct((N_PAD, 4 * HID), jnp.float32),
        mesh=mesh,
        scratch_types=[
            pltpu.VMEM((NPS + 24,), jnp.int32),
            pltpu.VMEM((MAXCH_PAD,), jnp.int32),
            pltpu.VMEM((C_EDGES,), jnp.int32),
            pltpu.VMEM((C_EDGES,), jnp.int32),
            pltpu.VMEM((C_EDGES, HID), jnp.float32),
            pltpu.VMEM((C_EDGES, HID), jnp.float32),
            pltpu.VMEM((64, 4 * HID), jnp.float32),
            pltpu.SemaphoreType.DMA,
            pltpu.SemaphoreType.DMA,
        ],
        interpret=interpret,
    )


def _dense_body(with_last, agg_ref, deg_ref, h_ref,
                w_ref, cb_ref, g_ref, bb_ref, fw_ref, fb_ref,
                wih_ref, whh_ref, bih_ref, bhh_ref, lw_ref, lb_ref,
                h_out_ref, y_out_ref):
    deg = deg_ref[:]
    degc = jnp.maximum(deg, 1.0)
    inv = 1.0 / degc
    a = agg_ref[:]
    has = deg > 0.0
    mean = jnp.where(has, a[:, :128] * inv, 0.0)
    var = jnp.maximum(a[:, 384:] * inv - mean * mean, 0.0)
    std = jnp.where(has, jnp.sqrt(var + 1e-5), STD_EPS_ROOT)
    mn = jnp.where(has, a[:, 128:256], 0.0)
    mx = jnp.where(has, a[:, 256:384], 0.0)
    agg = jnp.concatenate([mean, mn, mx, std], axis=1)
    p = jnp.dot(agg, w_ref[:], preferred_element_type=jnp.float32)
    logd = jnp.log(degc + 1.0)
    sc1 = logd * (1.0 / AVG_LOG)
    sc2 = AVG_LOG / logd
    x = p[:, :128] + sc1 * p[:, 128:256] + sc2 * p[:, 256:384] + cb_ref[:]
    x = x * (g_ref[:] * (1.0 / math.sqrt(1.0 + 1e-5))) + bb_ref[:]
    x = jnp.dot(x, fw_ref[:], preferred_element_type=jnp.float32) + fb_ref[:]
    x = jnp.maximum(x, 0.0)
    h = h_ref[:]
    gi = jnp.dot(x, wih_ref[:], preferred_element_type=jnp.float32) + bih_ref[:]
    gh = jnp.dot(h, whh_ref[:], preferred_element_type=jnp.float32) + bhh_ref[:]
    r = jax.nn.sigmoid(gi[:, :128] + gh[:, :128])
    z = jax.nn.sigmoid(gi[:, 128:256] + gh[:, 128:256])
    ng = jnp.tanh(gi[:, 256:] + r * gh[:, 256:])
    hn = (1.0 - z) * ng + z * h
    h_out_ref[:] = hn
    if with_last:
        y_out_ref[:] = (jnp.dot(hn, lw_ref[:], preferred_element_type=jnp.float32)
                        + lb_ref[:])
    else:
        y_out_ref[:] = hn


def _row_spec():
    return pl.BlockSpec((BLK, HID), lambda i: (i, 0))


def _full_spec(shape):
    nd = len(shape)
    return pl.BlockSpec(shape, lambda i, _nd=nd: (0,) * nd)


def _dense_layer(with_last, agg, deg, h, w_all, cb, g, bb, fw, fb,
                 wih, whh, bih, bhh, lw, lb):
    grid = (pl.cdiv(N_NODES, BLK),)
    in_specs = [
        pl.BlockSpec((BLK, 4 * HID), lambda i: (i, 0)),
        pl.BlockSpec((BLK, 1), lambda i: (i, 0)),
        _row_spec(),
        _full_spec((4 * HID, 3 * HID)), _full_spec((HID,)),
        _full_spec((HID,)), _full_spec((HID,)),
        _full_spec((HID, HID)), _full_spec((HID,)),
        _full_spec((HID, 3 * HID)), _full_spec((HID, 3 * HID)),
        _full_spec((3 * HID,)), _full_spec((3 * HID,)),
        _full_spec((HID, HID)), _full_spec((HID,)),
    ]
    out_specs = [_row_spec(), _row_spec()]
    out_shape = [jax.ShapeDtypeStruct((N_NODES, HID), jnp.float32),
                 jax.ShapeDtypeStruct((N_NODES, HID), jnp.float32)]
    fn = pl.pallas_call(
        functools.partial(_dense_body, with_last),
        grid=grid, in_specs=in_specs, out_specs=out_specs,
        out_shape=out_shape)
    return fn(agg, deg, h, w_all, cb, g, bb, fw, fb,
              wih, whh, bih, bhh, lw, lb)


def kernel(x, edge_index, batch, params):
    src = edge_index[0]
    dst = edge_index[1]

    perm = jnp.argsort(dst)
    ssrc = src[perm].astype(jnp.int32)
    sdst = dst[perm]
    offs = jnp.searchsorted(
        sdst, jnp.arange(N_NODES + 1, dtype=jnp.int32)).astype(jnp.int32)
    deg = (offs[1:] - offs[:-1]).astype(jnp.float32).reshape(N_NODES, 1)
    offs_pad = jnp.concatenate(
        [offs, jnp.full((OFFS_PAD - N_NODES - 1,), N_EDGES, jnp.int32)])
    src_pad = jnp.concatenate(
        [ssrc, jnp.zeros((SRC_PAD - N_EDGES,), jnp.int32)])
    nodes_lo = jnp.arange(NSUB, dtype=jnp.int32) * NPS
    e8_w = (offs[nodes_lo] // 8) * 8
    g1 = e8_w[:, None] + (jnp.arange(MAXCH_PAD, dtype=jnp.int32)[None, :] + 1) \
        * C_EDGES
    npe = jnp.searchsorted(offs[1:], g1, side='right').astype(jnp.int32)
    n_cnt_w = jnp.clip(N_NODES - nodes_lo, 0, NPS)
    chn = jnp.clip(npe - nodes_lo[:, None], 0, n_cnt_w[:, None])

    w_alls, cbs, gs, bbs, fws, fbs = [], [], [], [], [], []
    for i in range(3):
        w = params['conv%d_w' % i]
        wt = w.T
        w_all = jnp.concatenate([wt[:512], wt[512:1024], wt[1024:]], axis=1)
        w_alls.append(w_all)
        cbs.append(params['conv%d_b' % i])
        gs.append(params['bn%d_g' % i])
        bbs.append(params['bn%d_b' % i])
        fws.append(params['fc%d_w' % i].T)
        fbs.append(params['fc%d_b' % i])
    wih = params['gru_w_ih'].T
    whh = params['gru_w_hh'].T
    bih = params['gru_b_ih']
    bhh = params['gru_b_hh']
    lw = params['last_w'].T
    lb = params['last_b']

    h = jnp.zeros((N_NODES, HID), jnp.float32)
    cur = x
    y = None
    for i in range(3):
        agg = _make_sc_agg()(cur, src_pad, offs_pad, chn)
        with_last = (i == 2)
        h, y = _dense_layer(with_last, agg, deg, h,
                            w_alls[i], cbs[i], gs[i], bbs[i], fws[i], fbs[i],
                            wih, whh, bih, bhh, lw, lb)
        cur = h
    return y


# speedup vs baseline: 11.7869x; 11.7869x over previous
import functools
import math

import jax
import jax.numpy as jnp
import numpy as np
from jax import lax
from jax.experimental import pallas as pl
from jax.experimental.pallas import tpu as pltpu
from jax.experimental.pallas import tpu_sc as plsc

N_NODES = 10000
N_EDGES = 320000
HID = 128
AVG_LOG = float(np.log(33.0))
BLK = 512

NSUB = 32
NPS = 320
N_PAD = 10016
OFFS_PAD = 10264
C_EDGES = 128
SRC_PAD = N_EDGES + 256
MAXCH = (N_EDGES + C_EDGES + 8) // C_EDGES + 1
MAXCH_PAD = ((MAXCH + 23) // 8) * 8

STD_EPS_ROOT = float(math.sqrt(1e-5))


def _sc_agg_body(x_hbm, src_hbm, offs_hbm, chn_hbm, out_hbm,
                 offs_v, chn_v, idx_v0, idx_v1, rows_v0, rows_v1, stage,
                 sem0, sem1):
    w = lax.axis_index("s") * 2 + lax.axis_index("c")
    node_lo = w * NPS
    n_cnt = jnp.minimum(NPS, N_NODES - node_lo)
    pltpu.sync_copy(offs_hbm.at[pl.ds(node_lo, NPS + 24)], offs_v)
    pltpu.sync_copy(chn_hbm.at[w], chn_v)

    def offs_at(i):
        return offs_v[pl.ds(i, 16)]

    e_lo = offs_at(0)[0]
    e_hi = offs_at(n_cnt)[0]
    e8 = (e_lo // 8) * 8
    cnt = e_hi - e8
    nch = (cnt + C_EDGES - 1) // C_EDGES

    sems = (sem0, sem1)
    idxs = (idx_v0, idx_v1)
    rows = (rows_v0, rows_v1)

    def start(k, slot):
        pltpu.sync_copy(src_hbm.at[pl.ds(e8 + k * C_EDGES, C_EDGES)],
                        idxs[slot])
        pltpu.async_copy(x_hbm.at[idxs[slot]], rows[slot], sems[slot])

    def wait(slot):
        pltpu.make_async_copy(x_hbm.at[idxs[slot]], rows[slot],
                              sems[slot]).wait()

    def init_accs():
        z = jnp.zeros((16,), jnp.float32)
        ninf = jnp.full((16,), -jnp.inf, jnp.float32)
        pinf = jnp.full((16,), jnp.inf, jnp.float32)
        return (z,) * 8 + (z,) * 8 + (ninf,) * 8 + (pinf,) * 8

    def edge_loop(j0, j1, slot, accs):
        rv = rows[slot]

        def eb(j, a):
            out = []
            for c in range(8):
                r = rv[j, pl.ds(c * 16, 16)]
                out.append(a[c] + r)
            for c in range(8):
                r = rv[j, pl.ds(c * 16, 16)]
                out.append(a[8 + c] + r * r)
            for c in range(8):
                r = rv[j, pl.ds(c * 16, 16)]
                out.append(jnp.maximum(a[16 + c], r))
            for c in range(8):
                r = rv[j, pl.ds(c * 16, 16)]
                out.append(jnp.minimum(a[24 + c], r))
            return tuple(out)
        return lax.fori_loop(j0, jnp.maximum(j0, j1), eb, accs)

    def flush(np_, accs):
        row = np_ % 64
        for c in range(8):
            stage[row, pl.ds(c * 16, 16)] = accs[c]
            stage[row, pl.ds(128 + c * 16, 16)] = accs[24 + c]
            stage[row, pl.ds(256 + c * 16, 16)] = accs[16 + c]
            stage[row, pl.ds(384 + c * 16, 16)] = accs[8 + c]
        blk = np_ // 32

        @pl.when(np_ % 32 == 31)
        def _dma():
            pltpu.sync_copy(stage.at[pl.ds((blk % 2) * 32, 32)],
                            out_hbm.at[pl.ds(node_lo + blk * 32, 32)])

    def process(k, slot, carry):
        g0 = e8 + k * C_EDGES
        g1 = g0 + C_EDGES
        np0 = carry[0]
        np_end = chn_v[pl.ds(k, 16)][0]

        def nbody(np_, accs):
            v = offs_at(np_)
            a = jnp.maximum(v[0], g0)
            b = v[1]
            accs = edge_loop(a - g0, b - g0, slot, accs)
            flush(np_, accs)
            return init_accs()

        accs = lax.fori_loop(np0, jnp.maximum(np0, np_end), nbody, carry[1:])
        np_ = jnp.maximum(np0, np_end)
        v = offs_at(np_)
        a = jnp.maximum(v[0], g0)
        b = jnp.minimum(v[1], g1)
        b = jnp.where(np_ < n_cnt, b, a)
        accs = edge_loop(a - g0, b - g0, slot, accs)
        return (np_,) + accs

    @pl.when(nch > 0)
    def _prologue():
        start(0, 0)

    init = (jnp.int32(0),) + init_accs()

    def pair_body(p, carry):
        k0 = 2 * p

        @pl.when(k0 + 1 < nch)
        def _s1():
            start(k0 + 1, 1)

        wait(0)
        carry = process(k0, 0, carry)

        @pl.when(k0 + 2 < nch)
        def _s0():
            start(k0 + 2, 0)

        @pl.when(k0 + 1 < nch)
        def _w1():
            wait(1)

        carry = process(k0 + 1, 1, carry)
        return carry

    npairs = (nch + 1) // 2
    final = lax.fori_loop(0, npairs, pair_body, init)
    np_end = final[0]

    @pl.when(n_cnt % 32 != 0)
    def _tail_dma():
        blk = n_cnt // 32
        pltpu.sync_copy(stage.at[pl.ds((blk % 2) * 32, 32)],
                        out_hbm.at[pl.ds(node_lo + blk * 32, 32)])

    del np_end


@functools.cache
def _make_sc_agg(interpret=False):
    mesh = plsc.VectorSubcoreMesh(core_axis_name="c", subcore_axis_name="s",
                                  num_cores=2, num_subcores=16)
    return pl.kernel(
        _sc_agg_body,
        out_type=jax.ShapeDtypeStruct((N_PAD, 4 * HID), jnp.float32),
        mesh=mesh,
        scratch_types=[
            pltpu.VMEM((NPS + 24,), jnp.int32),
            pltpu.VMEM((MAXCH_PAD,), jnp.int32),
            pltpu.VMEM((C_EDGES,), jnp.int32),
            pltpu.VMEM((C_EDGES,), jnp.int32),
            pltpu.VMEM((C_EDGES, HID), jnp.float32),
            pltpu.VMEM((C_EDGES, HID), jnp.float32),
            pltpu.VMEM((64, 4 * HID), jnp.float32),
            pltpu.SemaphoreType.DMA,
            pltpu.SemaphoreType.DMA,
        ],
        interpret=interpret,
    )


def _dense_body(with_last, agg_ref, deg_ref, h_ref,
                w_ref, cb_ref, g_ref, bb_ref, fw_ref, fb_ref,
                wih_ref, whh_ref, bih_ref, bhh_ref, lw_ref, lb_ref,
                h_out_ref, y_out_ref):
    deg = deg_ref[:]
    degc = jnp.maximum(deg, 1.0)
    inv = 1.0 / degc
    a = agg_ref[:]
    has = deg > 0.0
    mean = jnp.where(has, a[:, :128] * inv, 0.0)
    var = jnp.maximum(a[:, 384:] * inv - mean * mean, 0.0)
    std = jnp.where(has, jnp.sqrt(var + 1e-5), STD_EPS_ROOT)
    mn = jnp.where(has, a[:, 128:256], 0.0)
    mx = jnp.where(has, a[:, 256:384], 0.0)
    agg = jnp.concatenate([mean, mn, mx, std], axis=1)
    p = jnp.dot(agg, w_ref[:], preferred_element_type=jnp.float32)
    logd = jnp.log(degc + 1.0)
    sc1 = logd * (1.0 / AVG_LOG)
    sc2 = AVG_LOG / logd
    x = p[:, :128] + sc1 * p[:, 128:256] + sc2 * p[:, 256:384] + cb_ref[:]
    x = x * (g_ref[:] * (1.0 / math.sqrt(1.0 + 1e-5))) + bb_ref[:]
    x = jnp.dot(x, fw_ref[:], preferred_element_type=jnp.float32) + fb_ref[:]
    x = jnp.maximum(x, 0.0)
    h = h_ref[:]
    gi = jnp.dot(x, wih_ref[:], preferred_element_type=jnp.float32) + bih_ref[:]
    gh = jnp.dot(h, whh_ref[:], preferred_element_type=jnp.float32) + bhh_ref[:]
    r = jax.nn.sigmoid(gi[:, :128] + gh[:, :128])
    z = jax.nn.sigmoid(gi[:, 128:256] + gh[:, 128:256])
    ng = jnp.tanh(gi[:, 256:] + r * gh[:, 256:])
    hn = (1.0 - z) * ng + z * h
    h_out_ref[:] = hn
    if with_last:
        y_out_ref[:] = (jnp.dot(hn, lw_ref[:], preferred_element_type=jnp.float32)
                        + lb_ref[:])
    else:
        y_out_ref[:] = hn


def _row_spec():
    return pl.BlockSpec((BLK, HID), lambda i: (i, 0))


def _full_spec(shape):
    nd = len(shape)
    return pl.BlockSpec(shape, lambda i, _nd=nd: (0,) * nd)


def _dense_layer(with_last, agg, deg, h, w_all, cb, g, bb, fw, fb,
                 wih, whh, bih, bhh, lw, lb):
    grid = (pl.cdiv(N_NODES, BLK),)
    in_specs = [
        pl.BlockSpec((BLK, 4 * HID), lambda i: (i, 0)),
        pl.BlockSpec((BLK, 1), lambda i: (i, 0)),
        _row_spec(),
        _full_spec((4 * HID, 3 * HID)), _full_spec((HID,)),
        _full_spec((HID,)), _full_spec((HID,)),
        _full_spec((HID, HID)), _full_spec((HID,)),
        _full_spec((HID, 3 * HID)), _full_spec((HID, 3 * HID)),
        _full_spec((3 * HID,)), _full_spec((3 * HID,)),
        _full_spec((HID, HID)), _full_spec((HID,)),
    ]
    out_specs = [_row_spec(), _row_spec()]
    out_shape = [jax.ShapeDtypeStruct((N_NODES, HID), jnp.float32),
                 jax.ShapeDtypeStruct((N_NODES, HID), jnp.float32)]
    fn = pl.pallas_call(
        functools.partial(_dense_body, with_last),
        grid=grid, in_specs=in_specs, out_specs=out_specs,
        out_shape=out_shape)
    return fn(agg, deg, h, w_all, cb, g, bb, fw, fb,
              wih, whh, bih, bhh, lw, lb)


def kernel(x, edge_index, batch, params):
    src = edge_index[0]
    dst = edge_index[1]

    key = dst * 16384 + src
    skey = jnp.sort(key)
    ssrc = (skey % 16384).astype(jnp.int32)
    deg_i = jnp.zeros((N_NODES,), jnp.int32).at[dst].add(1)
    offs = jnp.concatenate([jnp.zeros((1,), jnp.int32),
                            jnp.cumsum(deg_i, dtype=jnp.int32)])
    deg = deg_i.astype(jnp.float32).reshape(N_NODES, 1)
    offs_pad = jnp.concatenate(
        [offs, jnp.full((OFFS_PAD - N_NODES - 1,), N_EDGES, jnp.int32)])
    src_pad = jnp.concatenate(
        [ssrc, jnp.zeros((SRC_PAD - N_EDGES,), jnp.int32)])
    e_lo_w = offs[:NSUB * NPS - NPS + 1:NPS]
    e8_w = (e_lo_w // 8) * 8
    big = N_EDGES + C_EDGES * (MAXCH_PAD + 16)
    offs_end = jnp.concatenate(
        [offs[1:], jnp.full((NSUB * NPS - N_NODES,), big, jnp.int32)])
    c_np = (offs_end.reshape(NSUB, NPS) - 1 - e8_w[:, None]) // C_EDGES
    k_idx = jnp.arange(MAXCH_PAD, dtype=jnp.int32)
    chn = jnp.sum(c_np[:, None, :] <= k_idx[None, :, None], axis=2,
                  dtype=jnp.int32)

    w_alls, cbs, gs, bbs, fws, fbs = [], [], [], [], [], []
    for i in range(3):
        w = params['conv%d_w' % i]
        wt = w.T
        w_all = jnp.concatenate([wt[:512], wt[512:1024], wt[1024:]], axis=1)
        w_alls.append(w_all)
        cbs.append(params['conv%d_b' % i])
        gs.append(params['bn%d_g' % i])
        bbs.append(params['bn%d_b' % i])
        fws.append(params['fc%d_w' % i].T)
        fbs.append(params['fc%d_b' % i])
    wih = params['gru_w_ih'].T
    whh = params['gru_w_hh'].T
    bih = params['gru_b_ih']
    bhh = params['gru_b_hh']
    lw = params['last_w'].T
    lb = params['last_b']

    h = jnp.zeros((N_NODES, HID), jnp.float32)
    cur = x
    y = None
    for i in range(3):
        agg = _make_sc_agg()(cur, src_pad, offs_pad, chn)
        with_last = (i == 2)
        h, y = _dense_layer(with_last, agg, deg, h,
                            w_alls[i], cbs[i], gs[i], bbs[i], fws[i], fbs[i],
                            wih, whh, bih, bhh, lw, lb)
        cur = h
    return y
